# baseline (speedup 1.0000x reference)
import functools

import jax
import jax.numpy as jnp
from jax import lax
from jax.experimental import pallas as pl
from jax.experimental.pallas import tpu as pltpu
from jax.experimental.pallas import tpu_sc as plsc

NUM_ENTITIES = 1000000
NUM_RELATIONS = 1000
D_EMB = 64
BATCH = 16384
W = 128

NC = 2
NS = 16
L = 16
NW = NC * NS
BPW = BATCH // NW
CHUNK = 128
HALF = 256
NCHUNK = HALF // CHUNK


def _sqrt16(x):
    i = lax.bitcast_convert_type(x, jnp.int32)
    i = jnp.int32(0x5F3759DF) - lax.shift_right_arithmetic(i, jnp.int32(1))
    y = lax.bitcast_convert_type(i, jnp.float32)
    half = x * 0.5
    for _ in range(3):
        y = y * (1.5 - half * y * y)
    return jnp.where(x > 0.0, x * y, 0.0)


def _body(src_hbm, rel_hbm, tgt_hbm, ent_hbm, rtab_hbm, out_hbm,
          srow_v, spar_v, rrow_v, rpar_v, trow_v, tpar_v,
          h_v, r_v, t_v, o_v, sem):
    wid = lax.axis_index("s") * NC + lax.axis_index("c")
    base = wid * BPW

    for c in range(BPW // CHUNK):
        cbase = base + c * CHUNK
        pltpu.sync_copy(src_hbm.at[pl.ds(cbase, CHUNK)], srow_v.at[c])
        pltpu.sync_copy(tgt_hbm.at[pl.ds(cbase, CHUNK)], trow_v.at[c])
        pltpu.sync_copy(rel_hbm.at[pl.ds(cbase, CHUNK)], rrow_v.at[c])
    for c in range(BPW // CHUNK):
        for g in range(CHUNK // L):
            sl = pl.ds(g * L, L)
            for row_v, par_v in ((srow_v, spar_v), (trow_v, tpar_v),
                                 (rrow_v, rpar_v)):
                idx = row_v[c, sl]
                par_v[c, sl] = lax.shift_left(
                    jnp.bitwise_and(idx, jnp.int32(1)), jnp.int32(6))
                row_v[c, sl] = lax.shift_right_logical(idx, jnp.int32(1))

    lanes = lax.iota(jnp.int32, L)

    for half in range(BPW // HALF):
        hbase = half * NCHUNK
        copies = []
        for c in range(NCHUNK):
            sl = pl.ds(c * CHUNK, CHUNK)
            copies.append(pltpu.async_copy(
                ent_hbm.at[srow_v.at[hbase + c]], h_v.at[sl], sem))
            copies.append(pltpu.async_copy(
                ent_hbm.at[trow_v.at[hbase + c]], t_v.at[sl], sem))
            copies.append(pltpu.async_copy(
                rtab_hbm.at[rrow_v.at[hbase + c]], r_v.at[sl], sem))
        for cp in copies:
            cp.wait()

        def group(g, carry):
            i0 = g * L
            c = hbase + i0 // CHUNK
            gsl = pl.ds((i0 % CHUNK), L)
            ph_vec = spar_v[c, gsl]
            pt_vec = tpar_v[c, gsl]
            pr_vec = rpar_v[c, gsl]
            vec = jnp.zeros((L,), jnp.float32)
            for u in range(L):
                i = i0 + u
                ph = ph_vec[u]
                pt = pt_vec[u]
                pr = pr_vec[u]
                p = jnp.zeros((L,), jnp.float32)
                for q in range(D_EMB // L):
                    d = (h_v[i, pl.ds(ph + q * L, L)]
                         + r_v[i, pl.ds(pr + q * L, L)]
                         - t_v[i, pl.ds(pt + q * L, L)])
                    p = p + d * d
                vec = jnp.where(lanes == u, jnp.sum(p), vec)
            o_v[pl.ds(half * HALF + i0, L)] = _sqrt16(vec)
            return carry

        lax.fori_loop(0, HALF // L, group, 0)

    pltpu.sync_copy(o_v, out_hbm.at[pl.ds(base, BPW)])


_sc_call = functools.partial(
    pl.kernel,
    out_type=jax.ShapeDtypeStruct((BATCH,), jnp.float32),
    mesh=plsc.VectorSubcoreMesh(core_axis_name="c", subcore_axis_name="s"),
    scratch_types=[
        pltpu.VMEM((BPW // CHUNK, CHUNK), jnp.int32),
        pltpu.VMEM((BPW // CHUNK, CHUNK), jnp.int32),
        pltpu.VMEM((BPW // CHUNK, CHUNK), jnp.int32),
        pltpu.VMEM((BPW // CHUNK, CHUNK), jnp.int32),
        pltpu.VMEM((BPW // CHUNK, CHUNK), jnp.int32),
        pltpu.VMEM((BPW // CHUNK, CHUNK), jnp.int32),
        pltpu.VMEM((HALF, W), jnp.float32),
        pltpu.VMEM((HALF, W), jnp.float32),
        pltpu.VMEM((HALF, W), jnp.float32),
        pltpu.VMEM((BPW,), jnp.float32),
        pltpu.SemaphoreType.DMA,
    ],
    compiler_params=pltpu.CompilerParams(
        needs_layout_passes=False, use_tc_tiling_on_sc=False),
)(_body)


@jax.jit
def kernel(src, rel, tgt, entity_emb, relation_emb):
    src = src.astype(jnp.int32)
    rel = rel.astype(jnp.int32)
    tgt = tgt.astype(jnp.int32)
    ent2 = jnp.reshape(entity_emb, (NUM_ENTITIES // 2, W))
    rtab2 = jnp.reshape(relation_emb, (NUM_RELATIONS // 2, W))
    return _sc_call(src, rel, tgt, ent2, rtab2)


# speedup vs baseline: 1.4331x; 1.4331x over previous
import functools

import jax
import jax.numpy as jnp
from jax import lax
from jax.experimental import pallas as pl
from jax.experimental.pallas import tpu as pltpu
from jax.experimental.pallas import tpu_sc as plsc

NUM_ENTITIES = 1000000
NUM_RELATIONS = 1000
D_EMB = 64
BATCH = 16384

NC = 2
NS = 16
L = 16
NW = NC * NS
BPW = BATCH // NW
CSZ = 32
NSTEP = BPW // CSZ


def _sqrt16(x):
    i = lax.bitcast_convert_type(x, jnp.int32)
    i = jnp.int32(0x5F3759DF) - lax.shift_right_arithmetic(i, jnp.int32(1))
    y = lax.bitcast_convert_type(i, jnp.float32)
    half = x * 0.5
    for _ in range(3):
        y = y * (1.5 - half * y * y)
    return jnp.where(x > 0.0, x * y, 0.0)


def _body(src_hbm, rel_hbm, tgt_hbm, ent_hbm, rtab_hbm, out_hbm,
          src_v, rel_v, tgt_v, h_v, r_v, t_v, o_v, sem):
    wid = lax.axis_index("s") * NC + lax.axis_index("c")
    base = wid * BPW

    pltpu.sync_copy(src_hbm.at[pl.ds(base, BPW)], src_v)
    pltpu.sync_copy(tgt_hbm.at[pl.ds(base, BPW)], tgt_v)
    pltpu.sync_copy(rel_hbm.at[pl.ds(base, BPW)], rel_v)

    lanes = lax.iota(jnp.int32, L)

    def step(s, carry):
        i0 = s * CSZ
        idvecs = []
        for g in range(CSZ // L):
            sl = pl.ds(i0 + g * L, L)
            idvecs.append((src_v[sl], tgt_v[sl], rel_v[sl]))
        for g, (sv, tv, rv) in enumerate(idvecs):
            for u in range(L):
                l = g * L + u
                dst = pl.ds(8 * l, 8)
                sid = sv[u]
                tid = tv[u]
                rid = rv[u]
                stb = pl.multiple_of(jnp.bitwise_and(sid, jnp.int32(-8)), 8)
                ttb = pl.multiple_of(jnp.bitwise_and(tid, jnp.int32(-8)), 8)
                rtb = pl.multiple_of(jnp.bitwise_and(rid, jnp.int32(-8)), 8)
                pltpu.async_copy(ent_hbm.at[pl.ds(stb, 8), :], h_v.at[dst], sem)
                pltpu.async_copy(ent_hbm.at[pl.ds(ttb, 8), :], t_v.at[dst], sem)
                pltpu.async_copy(rtab_hbm.at[pl.ds(rtb, 8), :], r_v.at[dst], sem)
        pltpu.make_async_copy(ent_hbm.at[pl.ds(0, CSZ * 8), :], h_v, sem).wait()
        pltpu.make_async_copy(ent_hbm.at[pl.ds(0, CSZ * 8), :], t_v, sem).wait()
        pltpu.make_async_copy(ent_hbm.at[pl.ds(0, CSZ * 8), :], r_v, sem).wait()

        for g, (sv, tv, rv) in enumerate(idvecs):
            vec = jnp.zeros((L,), jnp.float32)
            for u in range(L):
                l = g * L + u
                ih = 8 * l + jnp.bitwise_and(sv[u], jnp.int32(7))
                it = 8 * l + jnp.bitwise_and(tv[u], jnp.int32(7))
                ir = 8 * l + jnp.bitwise_and(rv[u], jnp.int32(7))
                p = jnp.zeros((L,), jnp.float32)
                for q in range(D_EMB // L):
                    sl = pl.ds(q * L, L)
                    d = h_v[ih, sl] + r_v[ir, sl] - t_v[it, sl]
                    p = p + d * d
                vec = jnp.where(lanes == u, jnp.sum(p), vec)
            o_v[pl.ds(i0 + g * L, L)] = _sqrt16(vec)
        return carry

    lax.fori_loop(0, NSTEP, step, 0)

    pltpu.sync_copy(o_v, out_hbm.at[pl.ds(base, BPW)])


_sc_call = functools.partial(
    pl.kernel,
    out_type=jax.ShapeDtypeStruct((BATCH,), jnp.float32),
    mesh=plsc.VectorSubcoreMesh(core_axis_name="c", subcore_axis_name="s"),
    scratch_types=[
        pltpu.VMEM((BPW,), jnp.int32),
        pltpu.VMEM((BPW,), jnp.int32),
        pltpu.VMEM((BPW,), jnp.int32),
        pltpu.VMEM((CSZ * 8, D_EMB), jnp.float32),
        pltpu.VMEM((CSZ * 8, D_EMB), jnp.float32),
        pltpu.VMEM((CSZ * 8, D_EMB), jnp.float32),
        pltpu.VMEM((BPW,), jnp.float32),
        pltpu.SemaphoreType.DMA,
    ],
    compiler_params=pltpu.CompilerParams(
        needs_layout_passes=False, use_tc_tiling_on_sc=True),
)(_body)


@jax.jit
def kernel(src, rel, tgt, entity_emb, relation_emb):
    src = src.astype(jnp.int32)
    rel = rel.astype(jnp.int32)
    tgt = tgt.astype(jnp.int32)
    return _sc_call(src, rel, tgt, entity_emb, relation_emb)


# speedup vs baseline: 1.4505x; 1.0122x over previous
import functools

import jax
import jax.numpy as jnp
from jax import lax
from jax.experimental import pallas as pl
from jax.experimental.pallas import tpu as pltpu
from jax.experimental.pallas import tpu_sc as plsc

NUM_ENTITIES = 1000000
NUM_RELATIONS = 1000
D_EMB = 64
BATCH = 16384

NC = 2
NS = 16
L = 16
NW = NC * NS
BPW = BATCH // NW
CSZ = 16
NSTEP = BPW // CSZ


def _sqrt16(x):
    i = lax.bitcast_convert_type(x, jnp.int32)
    i = jnp.int32(0x5F3759DF) - lax.shift_right_arithmetic(i, jnp.int32(1))
    y = lax.bitcast_convert_type(i, jnp.float32)
    half = x * 0.5
    for _ in range(3):
        y = y * (1.5 - half * y * y)
    return jnp.where(x > 0.0, x * y, 0.0)


def _body(src_hbm, rel_hbm, tgt_hbm, ent_hbm, rtab_hbm, out_hbm,
          src_v, rel_v, tgt_v,
          h0, r0, t0, h1, r1, t1, o_v, sem0, sem1):
    wid = lax.axis_index("s") * NC + lax.axis_index("c")
    base = wid * BPW

    pltpu.sync_copy(src_hbm.at[pl.ds(base, BPW)], src_v)
    pltpu.sync_copy(tgt_hbm.at[pl.ds(base, BPW)], tgt_v)
    pltpu.sync_copy(rel_hbm.at[pl.ds(base, BPW)], rel_v)

    lanes = lax.iota(jnp.int32, L)
    bufs = ((h0, r0, t0, sem0), (h1, r1, t1, sem1))

    def ids_of(s):
        sl = pl.ds(s * CSZ, L)
        return src_v[sl], tgt_v[sl], rel_v[sl]

    def issue(s, which):
        h_v, r_v, t_v, sem = bufs[which]
        sv, tv, rv = ids_of(s)
        for u in range(L):
            dst = pl.ds(8 * u, 8)
            stb = pl.multiple_of(jnp.bitwise_and(sv[u], jnp.int32(-8)), 8)
            ttb = pl.multiple_of(jnp.bitwise_and(tv[u], jnp.int32(-8)), 8)
            rtb = pl.multiple_of(jnp.bitwise_and(rv[u], jnp.int32(-8)), 8)
            pltpu.async_copy(ent_hbm.at[pl.ds(stb, 8), :], h_v.at[dst], sem)
            pltpu.async_copy(ent_hbm.at[pl.ds(ttb, 8), :], t_v.at[dst], sem)
            pltpu.async_copy(rtab_hbm.at[pl.ds(rtb, 8), :], r_v.at[dst], sem)

    def drain(which):
        h_v, r_v, t_v, sem = bufs[which]
        pltpu.make_async_copy(ent_hbm.at[pl.ds(0, CSZ * 8), :], h_v, sem).wait()
        pltpu.make_async_copy(ent_hbm.at[pl.ds(0, CSZ * 8), :], t_v, sem).wait()
        pltpu.make_async_copy(ent_hbm.at[pl.ds(0, CSZ * 8), :], r_v, sem).wait()

    def compute(s, which):
        h_v, r_v, t_v, _ = bufs[which]
        sv, tv, rv = ids_of(s)
        vec = jnp.zeros((L,), jnp.float32)
        for u in range(L):
            ih = 8 * u + jnp.bitwise_and(sv[u], jnp.int32(7))
            it = 8 * u + jnp.bitwise_and(tv[u], jnp.int32(7))
            ir = 8 * u + jnp.bitwise_and(rv[u], jnp.int32(7))
            p = jnp.zeros((L,), jnp.float32)
            for q in range(D_EMB // L):
                sl = pl.ds(q * L, L)
                d = h_v[ih, sl] + r_v[ir, sl] - t_v[it, sl]
                p = p + d * d
            vec = jnp.where(lanes == u, jnp.sum(p), vec)
        o_v[pl.ds(s * CSZ, L)] = _sqrt16(vec)

    issue(0, 0)

    def pair(k, carry):
        s0 = 2 * k
        s1 = s0 + 1
        issue(s1, 1)
        drain(0)
        compute(s0, 0)
        s2 = jnp.minimum(s0 + 2, NSTEP - 2)
        issue(s2, 0)
        drain(1)
        compute(s1, 1)
        return carry

    lax.fori_loop(0, NSTEP // 2, pair, 0)
    drain(0)

    pltpu.sync_copy(o_v, out_hbm.at[pl.ds(base, BPW)])


_sc_call = functools.partial(
    pl.kernel,
    out_type=jax.ShapeDtypeStruct((BATCH,), jnp.float32),
    mesh=plsc.VectorSubcoreMesh(core_axis_name="c", subcore_axis_name="s"),
    scratch_types=[
        pltpu.VMEM((BPW,), jnp.int32),
        pltpu.VMEM((BPW,), jnp.int32),
        pltpu.VMEM((BPW,), jnp.int32),
        pltpu.VMEM((CSZ * 8, D_EMB), jnp.float32),
        pltpu.VMEM((CSZ * 8, D_EMB), jnp.float32),
        pltpu.VMEM((CSZ * 8, D_EMB), jnp.float32),
        pltpu.VMEM((CSZ * 8, D_EMB), jnp.float32),
        pltpu.VMEM((CSZ * 8, D_EMB), jnp.float32),
        pltpu.VMEM((CSZ * 8, D_EMB), jnp.float32),
        pltpu.VMEM((BPW,), jnp.float32),
        pltpu.SemaphoreType.DMA,
        pltpu.SemaphoreType.DMA,
    ],
    compiler_params=pltpu.CompilerParams(
        needs_layout_passes=False, use_tc_tiling_on_sc=True),
)(_body)


@jax.jit
def kernel(src, rel, tgt, entity_emb, relation_emb):
    src = src.astype(jnp.int32)
    rel = rel.astype(jnp.int32)
    tgt = tgt.astype(jnp.int32)
    return _sc_call(src, rel, tgt, entity_emb, relation_emb)


# speedup vs baseline: 1.5239x; 1.0506x over previous
import functools

import jax
import jax.numpy as jnp
from jax import lax
from jax.experimental import pallas as pl
from jax.experimental.pallas import tpu as pltpu
from jax.experimental.pallas import tpu_sc as plsc

NUM_ENTITIES = 1000000
NUM_RELATIONS = 1000
D_EMB = 64
BATCH = 16384

NC = 2
NS = 16
L = 16
NW = NC * NS
BPW = BATCH // NW
CSZ = 16
NSTEP = BPW // CSZ


def _sqrt16(x):
    i = lax.bitcast_convert_type(x, jnp.int32)
    i = jnp.int32(0x5F3759DF) - lax.shift_right_arithmetic(i, jnp.int32(1))
    y = lax.bitcast_convert_type(i, jnp.float32)
    half = x * 0.5
    for _ in range(3):
        y = y * (1.5 - half * y * y)
    return jnp.where(x > 0.0, x * y, 0.0)


def _body(src_hbm, rel_hbm, tgt_hbm, ent_hbm, rtab_hbm, out_hbm,
          src_v, rel_v, tgt_v, rrow_v,
          h0, r0, t0, h1, r1, t1, o_v, sem0, sem1):
    wid = lax.axis_index("s") * NC + lax.axis_index("c")
    base = wid * BPW

    pltpu.sync_copy(src_hbm.at[pl.ds(base, BPW)], src_v)
    pltpu.sync_copy(tgt_hbm.at[pl.ds(base, BPW)], tgt_v)
    pltpu.sync_copy(rel_hbm.at[pl.ds(base, BPW)], rel_v)
    for k in range(NSTEP):
        rrow_v[k, :] = lax.shift_right_logical(
            rel_v[pl.ds(k * CSZ, L)], jnp.int32(1))

    lanes = lax.iota(jnp.int32, L)
    bufs = ((h0, r0, t0, sem0), (h1, r1, t1, sem1))

    def ids_of(s):
        sl = pl.ds(s * CSZ, L)
        return src_v[sl], tgt_v[sl], rel_v[sl]

    def issue(s, which):
        h_v, r_v, t_v, sem = bufs[which]
        sv, tv, rv = ids_of(s)
        pltpu.async_copy(rtab_hbm.at[rrow_v.at[s]], r_v, sem)
        for u in range(L):
            dst = pl.ds(8 * u, 8)
            stb = pl.multiple_of(jnp.bitwise_and(sv[u], jnp.int32(-8)), 8)
            ttb = pl.multiple_of(jnp.bitwise_and(tv[u], jnp.int32(-8)), 8)
            pltpu.async_copy(ent_hbm.at[pl.ds(stb, 8), :], h_v.at[dst], sem)
            pltpu.async_copy(ent_hbm.at[pl.ds(ttb, 8), :], t_v.at[dst], sem)

    def drain(which):
        h_v, r_v, t_v, sem = bufs[which]
        pltpu.make_async_copy(ent_hbm.at[pl.ds(0, CSZ * 8), :], h_v, sem).wait()
        pltpu.make_async_copy(ent_hbm.at[pl.ds(0, CSZ * 8), :], t_v, sem).wait()
        pltpu.make_async_copy(rtab_hbm.at[pl.ds(0, CSZ)], r_v, sem).wait()

    def compute(s, which):
        h_v, r_v, t_v, _ = bufs[which]
        sv, tv, rv = ids_of(s)
        vec = jnp.zeros((L,), jnp.float32)
        for u in range(L):
            ih = 8 * u + jnp.bitwise_and(sv[u], jnp.int32(7))
            it = 8 * u + jnp.bitwise_and(tv[u], jnp.int32(7))
            rodd = jnp.bitwise_and(rv[u], jnp.int32(1)) == 1
            p = jnp.zeros((L,), jnp.float32)
            for q in range(D_EMB // L):
                sl = pl.ds(q * L, L)
                ra = r_v[u, pl.ds(q * L, L)]
                rb = r_v[u, pl.ds(D_EMB + q * L, L)]
                d = h_v[ih, sl] + jnp.where(rodd, rb, ra) - t_v[it, sl]
                p = p + d * d
            vec = jnp.where(lanes == u, jnp.sum(p), vec)
        o_v[pl.ds(s * CSZ, L)] = _sqrt16(vec)

    issue(0, 0)

    def pair(k, carry):
        s0 = 2 * k
        s1 = s0 + 1
        issue(s1, 1)
        drain(0)
        compute(s0, 0)
        s2 = jnp.minimum(s0 + 2, NSTEP - 2)
        issue(s2, 0)
        drain(1)
        compute(s1, 1)
        return carry

    lax.fori_loop(0, NSTEP // 2, pair, 0)
    drain(0)

    pltpu.sync_copy(o_v, out_hbm.at[pl.ds(base, BPW)])


_sc_call = functools.partial(
    pl.kernel,
    out_type=jax.ShapeDtypeStruct((BATCH,), jnp.float32),
    mesh=plsc.VectorSubcoreMesh(core_axis_name="c", subcore_axis_name="s"),
    scratch_types=[
        pltpu.VMEM((BPW,), jnp.int32),
        pltpu.VMEM((BPW,), jnp.int32),
        pltpu.VMEM((BPW,), jnp.int32),
        pltpu.VMEM((NSTEP, CSZ), jnp.int32),
        pltpu.VMEM((CSZ * 8, D_EMB), jnp.float32),
        pltpu.VMEM((CSZ, 2 * D_EMB), jnp.float32),
        pltpu.VMEM((CSZ * 8, D_EMB), jnp.float32),
        pltpu.VMEM((CSZ * 8, D_EMB), jnp.float32),
        pltpu.VMEM((CSZ, 2 * D_EMB), jnp.float32),
        pltpu.VMEM((CSZ * 8, D_EMB), jnp.float32),
        pltpu.VMEM((BPW,), jnp.float32),
        pltpu.SemaphoreType.DMA,
        pltpu.SemaphoreType.DMA,
    ],
    compiler_params=pltpu.CompilerParams(
        needs_layout_passes=False, use_tc_tiling_on_sc=True),
)(_body)


@jax.jit
def kernel(src, rel, tgt, entity_emb, relation_emb):
    src = src.astype(jnp.int32)
    rel = rel.astype(jnp.int32)
    tgt = tgt.astype(jnp.int32)
    rtab2 = jnp.reshape(relation_emb, (NUM_RELATIONS // 2, 2 * D_EMB))
    return _sc_call(src, rel, tgt, entity_emb, rtab2)
